# Initial kernel scaffold; baseline (speedup 1.0000x reference)
import jax
import jax.numpy as jnp
from jax.experimental import pallas as pl


def kernel(x, edge_index, W1, b1, W2, b2):
    raise NotImplementedError("write your pallas kernel here")


# speedup vs baseline: 3.3124x; 3.3124x over previous
import functools

import jax
import jax.numpy as jnp
from jax import lax
from jax.experimental import pallas as pl
from jax.experimental.pallas import tpu as pltpu
from jax.experimental.pallas import tpu_sc as plsc

N_NODES = 10000
N_EDGES = 320000
DIM = 128

NC, NS = 2, 16
NW = NC * NS
B_TOTAL = 2 * N_EDGES
B_PER_W = B_TOTAL // NW
CHUNK = 80
N_CHUNKS = B_PER_W // CHUNK


def _table_kernel(x_ref, w_ref, b_ref, t_ref):
    t_ref[...] = (
        jnp.dot(x_ref[...], w_ref[0], preferred_element_type=jnp.float32)
        + b_ref[0]
    )


def _build_table(x, W1, b1):
    wr = W1.reshape(2, DIM, DIM)
    bias = jnp.stack([b1, jnp.zeros_like(b1)]).reshape(2, 1, DIM)
    return pl.pallas_call(
        _table_kernel,
        grid=(2,),
        in_specs=[
            pl.BlockSpec((N_NODES, DIM), lambda i: (0, 0)),
            pl.BlockSpec((1, DIM, DIM), lambda i: (i, 0, 0)),
            pl.BlockSpec((1, 1, DIM), lambda i: (i, 0, 0)),
        ],
        out_specs=pl.BlockSpec((N_NODES, DIM), lambda i: (i, 0)),
        out_shape=jax.ShapeDtypeStruct((2 * N_NODES, DIM), jnp.float32),
    )(x, wr, bias)


def _sc_gather(table, idx3):
    mesh = plsc.VectorSubcoreMesh(
        core_axis_name="c", subcore_axis_name="s", num_cores=NC, num_subcores=NS
    )

    @functools.partial(
        pl.kernel,
        out_type=jax.ShapeDtypeStruct((B_TOTAL, DIM), jnp.float32),
        mesh=mesh,
        scratch_types=[
            pltpu.VMEM((N_CHUNKS, CHUNK), jnp.int32),
            pltpu.VMEM((2, CHUNK, DIM), jnp.float32),
            pltpu.SemaphoreType.DMA,
            pltpu.SemaphoreType.DMA,
            pltpu.SemaphoreType.DMA,
            pltpu.SemaphoreType.DMA,
            pltpu.SemaphoreType.DMA,
        ],
    )
    def k(t_hbm, i_hbm, o_hbm, idx_v, rows_v, isem, gsem0, gsem1, osem0, osem1):
        wid = lax.axis_index("s") * NC + lax.axis_index("c")
        base = wid * B_PER_W
        gsems = (gsem0, gsem1)
        osems = (osem0, osem1)

        pltpu.async_copy(i_hbm.at[wid], idx_v, isem).wait()

        def gather_copy(ci, b):
            return pltpu.make_async_copy(
                t_hbm.at[idx_v.at[ci]], rows_v.at[b], gsems[b]
            )

        def out_copy(ci, b):
            return pltpu.make_async_copy(
                rows_v.at[b], o_hbm.at[pl.ds(base + ci * CHUNK, CHUNK)], osems[b]
            )

        gather_copy(0, 0).start()

        @pl.loop(0, N_CHUNKS, step=2)
        def _(i0):
            gather_copy(i0, 0).wait()
            out_copy(i0, 0).start()

            @pl.when(i0 > 0)
            def _():
                out_copy(i0 - 1, 1).wait()

            gather_copy(i0 + 1, 1).start()

            gather_copy(i0 + 1, 1).wait()
            out_copy(i0 + 1, 1).start()

            @pl.when(i0 + 2 < N_CHUNKS)
            def _():
                out_copy(i0, 0).wait()
                gather_copy(i0 + 2, 0).start()

        out_copy(N_CHUNKS - 2, 0).wait()
        out_copy(N_CHUNKS - 1, 1).wait()

    return k(table, idx3)


def _mlp_kernel(a_ref, b_ref, w_ref, bias_ref, o_ref):
    h = jnp.maximum(a_ref[0] + b_ref[0], 0.0)
    o_ref[...] = (
        jnp.dot(h, w_ref[...], preferred_element_type=jnp.float32)
        + bias_ref[...]
    )


def _mlp_out(g3, W2, b2):
    blk = 6400
    grid = (N_EDGES // blk,)
    return pl.pallas_call(
        _mlp_kernel,
        grid=grid,
        in_specs=[
            pl.BlockSpec((1, blk, DIM), lambda i: (0, i, 0)),
            pl.BlockSpec((1, blk, DIM), lambda i: (1, i, 0)),
            pl.BlockSpec((DIM, DIM), lambda i: (0, 0)),
            pl.BlockSpec((1, DIM), lambda i: (0, 0)),
        ],
        out_specs=pl.BlockSpec((blk, DIM), lambda i: (i, 0)),
        out_shape=jax.ShapeDtypeStruct((N_EDGES, DIM), jnp.float32),
    )(g3, g3, W2, b2.reshape(1, DIM))


def kernel(x, edge_index, W1, b1, W2, b2):
    x2 = x.reshape(-1, x.shape[-1])
    ei = edge_index.reshape(2, -1).astype(jnp.int32)
    table = _build_table(x2, W1, b1)
    idx = jnp.concatenate([ei[0], ei[1] + N_NODES])
    idx3 = idx.reshape(NW, N_CHUNKS, CHUNK)
    g = _sc_gather(table, idx3)
    out = _mlp_out(g.reshape(2, N_EDGES, DIM), W2, b2)
    return out.reshape(1, N_EDGES, DIM)


# speedup vs baseline: 5.3012x; 1.6004x over previous
import functools

import jax
import jax.numpy as jnp
from jax import lax
from jax.experimental import pallas as pl
from jax.experimental.pallas import tpu as pltpu
from jax.experimental.pallas import tpu_sc as plsc

N_NODES = 10000
N_EDGES = 320000
DIM = 128
HDIM = 64

NC, NS = 2, 16
NW = NC * NS
B_TOTAL = 2 * N_EDGES
B_PER_W = B_TOTAL // NW
CHUNK = 80
N_CHUNKS = B_PER_W // CHUNK
STAGE_ROWS = 1000


def _round_bf16_bits(u):
    return (u + 0x7FFF + ((u >> 16) & 1)) & jnp.uint32(0xFFFF0000)


def _pack_cols(t):
    lo = lax.bitcast_convert_type(t[:, :HDIM], jnp.uint32)
    hi = lax.bitcast_convert_type(t[:, HDIM:], jnp.uint32)
    return (_round_bf16_bits(lo) >> 16) | _round_bf16_bits(hi)


def _table_kernel(x_ref, w_ref, b_ref, t_ref):
    x = x_ref[...]
    ts = jnp.dot(x, w_ref[0], preferred_element_type=jnp.float32) + b_ref[0]
    te = jnp.dot(x, w_ref[1], preferred_element_type=jnp.float32)
    t_ref[...] = jnp.concatenate([_pack_cols(ts), _pack_cols(te)], axis=1)


def _build_table(x, W1, b1):
    wr = W1.reshape(2, DIM, DIM)
    blk = 1000
    return pl.pallas_call(
        _table_kernel,
        grid=(N_NODES // blk,),
        in_specs=[
            pl.BlockSpec((blk, DIM), lambda i: (i, 0)),
            pl.BlockSpec((2, DIM, DIM), lambda i: (0, 0, 0)),
            pl.BlockSpec((1, DIM), lambda i: (0, 0)),
        ],
        out_specs=pl.BlockSpec((blk, DIM), lambda i: (i, 0)),
        out_shape=jax.ShapeDtypeStruct((N_NODES, DIM), jnp.uint32),
    )(x, wr, b1.reshape(1, DIM))


def _sc_gather(table, idx3):
    mesh = plsc.VectorSubcoreMesh(
        core_axis_name="c", subcore_axis_name="s", num_cores=NC, num_subcores=NS
    )

    @functools.partial(
        pl.kernel,
        out_type=jax.ShapeDtypeStruct((B_TOTAL, DIM), jnp.uint32),
        mesh=mesh,
        scratch_types=[
            pltpu.VMEM((B_PER_W,), jnp.int32),
            pltpu.VMEM((2, CHUNK, DIM), jnp.uint32),
            pltpu.VMEM_SHARED((N_NODES, DIM), jnp.uint32),
            pltpu.SemaphoreType.DMA,
            pltpu.SemaphoreType.DMA,
            pltpu.SemaphoreType.DMA,
            pltpu.SemaphoreType.DMA,
            pltpu.SemaphoreType.DMA,
        ],
    )
    def k(t_hbm, i_hbm, o_hbm, idx_v, rows_v, t_sp, isem, gsem0, gsem1,
          osem0, osem1):
        cid = lax.axis_index("c")
        sid = lax.axis_index("s")
        wid = sid * NC + cid
        base = wid * B_PER_W
        gsems = (gsem0, gsem1)
        osems = (osem0, osem1)

        @pl.when(sid < N_NODES // STAGE_ROWS)
        def _():
            pltpu.sync_copy(
                t_hbm.at[pl.ds(sid * STAGE_ROWS, STAGE_ROWS)],
                t_sp.at[pl.ds(sid * STAGE_ROWS, STAGE_ROWS)],
            )

        plsc.subcore_barrier()

        pltpu.async_copy(i_hbm.at[pl.ds(base, B_PER_W)], idx_v, isem).wait()

        def gather_copy(ci, b):
            return pltpu.make_async_copy(
                t_sp.at[idx_v.at[pl.ds(ci * CHUNK, CHUNK)]], rows_v.at[b],
                gsems[b],
            )

        def out_copy(ci, b):
            return pltpu.make_async_copy(
                rows_v.at[b],
                o_hbm.at[pl.ds(base + ci * CHUNK, CHUNK)],
                osems[b],
            )

        gather_copy(0, 0).start()

        @pl.loop(0, N_CHUNKS, step=2)
        def _(i0):
            gather_copy(i0, 0).wait()
            out_copy(i0, 0).start()

            @pl.when(i0 > 0)
            def _():
                out_copy(i0 - 1, 1).wait()

            gather_copy(i0 + 1, 1).start()

            gather_copy(i0 + 1, 1).wait()
            out_copy(i0 + 1, 1).start()

            @pl.when(i0 + 2 < N_CHUNKS)
            def _():
                out_copy(i0, 0).wait()
                gather_copy(i0 + 2, 0).start()

        out_copy(N_CHUNKS - 2, 0).wait()
        out_copy(N_CHUNKS - 1, 1).wait()

    return k(table, idx3)


def _unpack(u):
    lo = lax.bitcast_convert_type(u << 16, jnp.float32)
    hi = lax.bitcast_convert_type(u & jnp.uint32(0xFFFF0000), jnp.float32)
    return lo, hi


def _mlp_kernel(a_ref, b_ref, w_ref, bias_ref, o_ref):
    a_lo, a_hi = _unpack(a_ref[0][:, :HDIM])
    b_lo, b_hi = _unpack(b_ref[0][:, HDIM:])
    h = jnp.concatenate([a_lo + b_lo, a_hi + b_hi], axis=1)
    h = jnp.maximum(h, 0.0)
    o_ref[...] = (
        jnp.dot(h, w_ref[...], preferred_element_type=jnp.float32)
        + bias_ref[...]
    )


def _mlp_out(g3, W2, b2):
    blk = 6400
    grid = (N_EDGES // blk,)
    return pl.pallas_call(
        _mlp_kernel,
        grid=grid,
        in_specs=[
            pl.BlockSpec((1, blk, DIM), lambda i: (0, i, 0)),
            pl.BlockSpec((1, blk, DIM), lambda i: (1, i, 0)),
            pl.BlockSpec((DIM, DIM), lambda i: (0, 0)),
            pl.BlockSpec((1, DIM), lambda i: (0, 0)),
        ],
        out_specs=pl.BlockSpec((blk, DIM), lambda i: (i, 0)),
        out_shape=jax.ShapeDtypeStruct((N_EDGES, DIM), jnp.float32),
    )(g3, g3, W2, b2.reshape(1, DIM))


def kernel(x, edge_index, W1, b1, W2, b2):
    x2 = x.reshape(-1, x.shape[-1])
    ei = edge_index.reshape(2, -1).astype(jnp.int32)
    table = _build_table(x2, W1, b1)
    g = _sc_gather(table, ei.reshape(B_TOTAL))
    out = _mlp_out(g.reshape(2, N_EDGES, DIM), W2, b2)
    return out.reshape(1, N_EDGES, DIM)
